# baseline (speedup 1.0000x reference)
import functools

import jax
import jax.numpy as jnp
from jax import lax
from jax.experimental import pallas as pl
from jax.experimental.pallas import tpu as pltpu
from jax.experimental.pallas import tpu_sc as plsc

BLOCK_ROWS = 2048
N_EXPERTS = 16
TOPK = 2
NC, NS, LANES = 2, 16, 16
NW = NC * NS
N_CHUNKS = 2


def _top2_block(x_ref, w_ref, b_ref, vals_ref, idx_ref):
    logits = jnp.dot(x_ref[...], w_ref[...],
                     preferred_element_type=jnp.float32) + b_ref[...]
    rows = logits.shape[0]
    lane = lax.broadcasted_iota(jnp.int32, (rows, N_EXPERTS), 1)

    m1 = jnp.max(logits, axis=1, keepdims=True)
    i1 = jnp.min(jnp.where(logits == m1, lane, N_EXPERTS), axis=1,
                 keepdims=True)
    masked = jnp.where(lane == i1, -jnp.inf, logits)
    m2 = jnp.max(masked, axis=1, keepdims=True)
    i2 = jnp.min(jnp.where(masked == m2, lane, N_EXPERTS), axis=1,
                 keepdims=True)

    pair = lax.broadcasted_iota(jnp.int32, (rows, TOPK), 1)
    vals_ref[...] = jnp.where(pair == 0, m1, m2)
    idx_ref[...] = jnp.where(pair == 0, i1, i2)


def _tc_top2(x, W, b2, chunk, rows):
    blk0 = chunk * (rows // BLOCK_ROWS)
    return pl.pallas_call(
        _top2_block,
        grid=(rows // BLOCK_ROWS,),
        in_specs=[
            pl.BlockSpec((BLOCK_ROWS, x.shape[1]), lambda i: (blk0 + i, 0)),
            pl.BlockSpec((x.shape[1], N_EXPERTS), lambda i: (0, 0)),
            pl.BlockSpec((1, N_EXPERTS), lambda i: (0, 0)),
        ],
        out_specs=[
            pl.BlockSpec((BLOCK_ROWS, TOPK), lambda i: (i, 0)),
            pl.BlockSpec((BLOCK_ROWS, TOPK), lambda i: (i, 0)),
        ],
        out_shape=[
            jax.ShapeDtypeStruct((rows, TOPK), jnp.float32),
            jax.ShapeDtypeStruct((rows, TOPK), jnp.int32),
        ],
    )(x, W, b2)


def _sc_scatter(vals_flat, idx_flat, rows):
    rpw = rows // NW
    groups = rpw // 8
    mesh = plsc.VectorSubcoreMesh(core_axis_name="c", subcore_axis_name="s",
                                  num_cores=NC, num_subcores=NS)

    @functools.partial(
        pl.kernel,
        out_type=jax.ShapeDtypeStruct((rows, N_EXPERTS), jnp.float32),
        mesh=mesh,
        scratch_types=[
            pltpu.VMEM((rpw * TOPK,), jnp.float32),
            pltpu.VMEM((rpw * TOPK,), jnp.int32),
            pltpu.VMEM((rpw, N_EXPERTS), jnp.float32),
        ],
    )
    def scatter(vals_hbm, idx_hbm, gates_hbm, vals_v, idx_v, gates_v):
        wid = lax.axis_index("s") * NC + lax.axis_index("c")
        base = wid * rpw
        pltpu.sync_copy(vals_hbm.at[pl.ds(base * TOPK, rpw * TOPK)], vals_v)
        pltpu.sync_copy(idx_hbm.at[pl.ds(base * TOPK, rpw * TOPK)], idx_v)
        lane = lax.iota(jnp.int32, LANES)
        zero = jnp.zeros((LANES,), jnp.float32)

        dnums = lax.GatherDimensionNumbers(offset_dims=(),
                                           collapsed_slice_dims=(0,),
                                           start_index_map=(0,))

        def shuffle(v, idx):
            return lax.gather(v, idx[:, None], dnums, (1,),
                              mode=lax.GatherScatterMode.PROMISE_IN_BOUNDS)

        def group_body(g, _):
            vv = vals_v[pl.ds(g * 16, 16)]
            iv = idx_v[pl.ds(g * 16, 16)]
            other = shuffle(vv, lane ^ 1)
            gv = 1.0 / (1.0 + jnp.exp(other - vv))
            for r in range(8):
                i1s = shuffle(iv, jnp.full((LANES,), 2 * r, jnp.int32))
                i2s = shuffle(iv, jnp.full((LANES,), 2 * r + 1, jnp.int32))
                g1s = shuffle(gv, jnp.full((LANES,), 2 * r, jnp.int32))
                g2s = shuffle(gv, jnp.full((LANES,), 2 * r + 1, jnp.int32))
                gates_v[g * 8 + r] = jnp.where(lane == i1s, g1s,
                                               jnp.where(lane == i2s, g2s,
                                                         zero))
            return 0

        lax.fori_loop(0, groups, group_body, 0)
        pltpu.sync_copy(gates_v, gates_hbm.at[pl.ds(base, rpw)])

    return scatter(vals_flat, idx_flat)


@jax.jit
def kernel(x, W, b):
    x = x.astype(jnp.float32)
    Wf = W.astype(jnp.float32)
    b2 = b.reshape(1, N_EXPERTS).astype(jnp.float32)
    rows = x.shape[0]
    h = rows // N_CHUNKS
    gates_parts, idx_parts = [], []
    for c in range(N_CHUNKS):
        vals, idx = _tc_top2(x, Wf, b2, c, h)
        gates_parts.append(_sc_scatter(vals.reshape(-1), idx.reshape(-1), h))
        idx_parts.append(idx)
    if N_CHUNKS == 1:
        return gates_parts[0], idx_parts[0]
    return (jnp.concatenate(gates_parts, axis=0),
            jnp.concatenate(idx_parts, axis=0))


# speedup vs baseline: 1.0045x; 1.0045x over previous
import functools

import jax
import jax.numpy as jnp
from jax import lax
from jax.experimental import pallas as pl
from jax.experimental.pallas import tpu as pltpu
from jax.experimental.pallas import tpu_sc as plsc

BLOCK_ROWS = 2048
N_EXPERTS = 16
TOPK = 2
NC, NS, LANES = 2, 16, 16
NW = NC * NS
NCHUNKS = 4


def _logits_block(x_ref, w_ref, b_ref, logits_ref):
    logits_ref[...] = jnp.dot(x_ref[...], w_ref[...],
                              preferred_element_type=jnp.float32) + b_ref[...]


def _tc_logits_chunk(x, W, b2, chunk, chunk_rows):
    blocks = chunk_rows // BLOCK_ROWS
    base_block = chunk * blocks
    return pl.pallas_call(
        _logits_block,
        grid=(blocks,),
        in_specs=[
            pl.BlockSpec((BLOCK_ROWS, x.shape[1]),
                         lambda i: (base_block + i, 0)),
            pl.BlockSpec((x.shape[1], N_EXPERTS), lambda i: (0, 0)),
            pl.BlockSpec((1, N_EXPERTS), lambda i: (0, 0)),
        ],
        out_specs=pl.BlockSpec((BLOCK_ROWS, N_EXPERTS), lambda i: (i, 0)),
        out_shape=jax.ShapeDtypeStruct((chunk_rows, N_EXPERTS), jnp.float32),
    )(x, W, b2)


def _sc_route(logits):
    rows = logits.shape[0]
    rpw = rows // NW
    groups = rpw // 8
    mesh = plsc.VectorSubcoreMesh(core_axis_name="c", subcore_axis_name="s",
                                  num_cores=NC, num_subcores=NS)

    @functools.partial(
        pl.kernel,
        out_type=[
            jax.ShapeDtypeStruct((rows, N_EXPERTS), jnp.float32),
            jax.ShapeDtypeStruct((rows * TOPK,), jnp.int32),
        ],
        mesh=mesh,
        scratch_types=[
            pltpu.VMEM((rpw, N_EXPERTS), jnp.float32),
            pltpu.VMEM((rpw * TOPK,), jnp.int32),
        ],
    )
    def route(logits_hbm, gates_hbm, idx_hbm, lg_v, idx_v):
        wid = lax.axis_index("s") * NC + lax.axis_index("c")
        base = wid * rpw
        pltpu.sync_copy(logits_hbm.at[pl.ds(base, rpw)], lg_v)
        lane = lax.iota(jnp.int32, 16)
        neg_inf = jnp.full((16,), -jnp.inf, jnp.float32)
        zero = jnp.zeros((16,), jnp.float32)

        dnums = lax.GatherDimensionNumbers(offset_dims=(),
                                           collapsed_slice_dims=(0,),
                                           start_index_map=(0,))

        def shuffle(v, idx):
            return lax.gather(v, idx[:, None], dnums, (1,),
                              mode=lax.GatherScatterMode.PROMISE_IN_BOUNDS)

        def lane_max(v):
            for k in (1, 2, 4, 8):
                v = jnp.maximum(v, shuffle(v, lane ^ k))
            return v

        def lane_min(v):
            for k in (1, 2, 4, 8):
                v = jnp.minimum(v, shuffle(v, lane ^ k))
            return v

        def argmax_first(v, m):
            return lane_min(jnp.where(v == m, lane, N_EXPERTS))

        def group_body(g, _):
            acc = jnp.zeros((16,), jnp.int32)
            for r in range(8):
                i = g * 8 + r
                v = lg_v[i]
                m1 = lane_max(v)
                i1 = argmax_first(v, m1)
                masked = jnp.where(lane == i1, neg_inf, v)
                m2 = lane_max(masked)
                i2 = argmax_first(masked, m2)
                e = jnp.exp(m2 - m1)
                g2 = e / (1.0 + e)
                g1 = 1.0 - g2
                lg_v[i] = jnp.where(lane == i1, g1,
                                    jnp.where(lane == i2, g2, zero))
                acc = jnp.where(lane == 2 * r, i1, acc)
                acc = jnp.where(lane == 2 * r + 1, i2, acc)
            idx_v[pl.ds(g * 16, 16)] = acc
            return 0

        lax.fori_loop(0, groups, group_body, 0)
        pltpu.sync_copy(lg_v, gates_hbm.at[pl.ds(base, rpw)])
        pltpu.sync_copy(idx_v, idx_hbm.at[pl.ds(base * TOPK, rpw * TOPK)])

    gates, idx_flat = route(logits)
    return gates, idx_flat


@jax.jit
def kernel(x, W, b):
    x = x.astype(jnp.float32)
    Wf = W.astype(jnp.float32)
    rows = x.shape[0]
    b2 = b.reshape(1, N_EXPERTS).astype(jnp.float32)
    chunk_rows = rows // NCHUNKS
    gates_parts, idx_parts = [], []
    for c in range(NCHUNKS):
        logits_c = _tc_logits_chunk(x, Wf, b2, c, chunk_rows)
        gates_c, idx_c = _sc_route(logits_c)
        gates_parts.append(gates_c)
        idx_parts.append(idx_c)
    gates = jnp.concatenate(gates_parts, axis=0)
    idx = jnp.concatenate(idx_parts, axis=0).reshape(rows, TOPK)
    return gates, idx


# speedup vs baseline: 1.0059x; 1.0014x over previous
import functools

import jax
import jax.numpy as jnp
from jax import lax
from jax.experimental import pallas as pl
from jax.experimental.pallas import tpu as pltpu
from jax.experimental.pallas import tpu_sc as plsc

BLOCK_ROWS = 2048
N_EXPERTS = 16
TOPK = 2
NC, NS, LANES = 2, 16, 16
NW = NC * NS
CHUNK_ROWS = (12288, 4096)


def _logits_block(x_ref, w_ref, b_ref, logits_ref):
    logits_ref[...] = jnp.dot(x_ref[...], w_ref[...],
                              preferred_element_type=jnp.float32) + b_ref[...]


def _tc_logits_chunk(x, W, b2, base_row, chunk_rows):
    blocks = chunk_rows // BLOCK_ROWS
    base_block = base_row // BLOCK_ROWS
    return pl.pallas_call(
        _logits_block,
        grid=(blocks,),
        in_specs=[
            pl.BlockSpec((BLOCK_ROWS, x.shape[1]),
                         lambda i: (base_block + i, 0)),
            pl.BlockSpec((x.shape[1], N_EXPERTS), lambda i: (0, 0)),
            pl.BlockSpec((1, N_EXPERTS), lambda i: (0, 0)),
        ],
        out_specs=pl.BlockSpec((BLOCK_ROWS, N_EXPERTS), lambda i: (i, 0)),
        out_shape=jax.ShapeDtypeStruct((chunk_rows, N_EXPERTS), jnp.float32),
    )(x, W, b2)


def _sc_route(logits):
    rows = logits.shape[0]
    rpw = rows // NW
    groups = rpw // 8
    mesh = plsc.VectorSubcoreMesh(core_axis_name="c", subcore_axis_name="s",
                                  num_cores=NC, num_subcores=NS)

    @functools.partial(
        pl.kernel,
        out_type=[
            jax.ShapeDtypeStruct((rows, N_EXPERTS), jnp.float32),
            jax.ShapeDtypeStruct((rows * TOPK,), jnp.int32),
        ],
        mesh=mesh,
        scratch_types=[
            pltpu.VMEM((rpw, N_EXPERTS), jnp.float32),
            pltpu.VMEM((rpw * TOPK,), jnp.int32),
        ],
    )
    def route(logits_hbm, gates_hbm, idx_hbm, lg_v, idx_v):
        wid = lax.axis_index("s") * NC + lax.axis_index("c")
        base = wid * rpw
        pltpu.sync_copy(logits_hbm.at[pl.ds(base, rpw)], lg_v)
        lane = lax.iota(jnp.int32, 16)
        neg_inf = jnp.full((16,), -jnp.inf, jnp.float32)
        zero = jnp.zeros((16,), jnp.float32)

        dnums = lax.GatherDimensionNumbers(offset_dims=(),
                                           collapsed_slice_dims=(0,),
                                           start_index_map=(0,))

        def shuffle(v, idx):
            return lax.gather(v, idx[:, None], dnums, (1,),
                              mode=lax.GatherScatterMode.PROMISE_IN_BOUNDS)

        def lane_max(v):
            for k in (1, 2, 4, 8):
                v = jnp.maximum(v, shuffle(v, lane ^ k))
            return v

        def lane_min(v):
            for k in (1, 2, 4, 8):
                v = jnp.minimum(v, shuffle(v, lane ^ k))
            return v

        def argmax_first(v, m):
            return lane_min(jnp.where(v == m, lane, N_EXPERTS))

        def group_body(g, _):
            acc = jnp.zeros((16,), jnp.int32)
            for r in range(8):
                i = g * 8 + r
                v = lg_v[i]
                m1 = lane_max(v)
                i1 = argmax_first(v, m1)
                masked = jnp.where(lane == i1, neg_inf, v)
                m2 = lane_max(masked)
                i2 = argmax_first(masked, m2)
                e = jnp.exp(m2 - m1)
                g2 = e / (1.0 + e)
                g1 = 1.0 - g2
                lg_v[i] = jnp.where(lane == i1, g1,
                                    jnp.where(lane == i2, g2, zero))
                acc = jnp.where(lane == 2 * r, i1, acc)
                acc = jnp.where(lane == 2 * r + 1, i2, acc)
            idx_v[pl.ds(g * 16, 16)] = acc
            return 0

        lax.fori_loop(0, groups, group_body, 0)
        pltpu.sync_copy(lg_v, gates_hbm.at[pl.ds(base, rpw)])
        pltpu.sync_copy(idx_v, idx_hbm.at[pl.ds(base * TOPK, rpw * TOPK)])

    gates, idx_flat = route(logits)
    return gates, idx_flat


@jax.jit
def kernel(x, W, b):
    x = x.astype(jnp.float32)
    Wf = W.astype(jnp.float32)
    rows = x.shape[0]
    b2 = b.reshape(1, N_EXPERTS).astype(jnp.float32)
    gates_parts, idx_parts = [], []
    base_row = 0
    for chunk_rows in CHUNK_ROWS:
        logits_c = _tc_logits_chunk(x, Wf, b2, base_row, chunk_rows)
        gates_c, idx_c = _sc_route(logits_c)
        gates_parts.append(gates_c)
        idx_parts.append(idx_c)
        base_row += chunk_rows
    gates = jnp.concatenate(gates_parts, axis=0)
    idx = jnp.concatenate(idx_parts, axis=0).reshape(rows, TOPK)
    return gates, idx


# speedup vs baseline: 1.0718x; 1.0655x over previous
import functools

import jax
import jax.numpy as jnp
from jax import lax
from jax.experimental import pallas as pl
from jax.experimental.pallas import tpu as pltpu
from jax.experimental.pallas import tpu_sc as plsc

BLOCK_ROWS = 2048
N_EXPERTS = 16
TOPK = 2
NC, NS, LANES = 2, 16, 16
NW = NC * NS
CHUNK_ROWS = (8192, 8192)


def _logits_block(x_ref, w_ref, b_ref, logits_ref):
    logits_ref[...] = jnp.dot(x_ref[...], w_ref[...],
                              preferred_element_type=jnp.float32) + b_ref[...]


def _tc_logits_chunk(x, W, b2, base_row, chunk_rows):
    blocks = chunk_rows // BLOCK_ROWS
    base_block = base_row // BLOCK_ROWS
    return pl.pallas_call(
        _logits_block,
        grid=(blocks,),
        in_specs=[
            pl.BlockSpec((BLOCK_ROWS, x.shape[1]),
                         lambda i: (base_block + i, 0)),
            pl.BlockSpec((x.shape[1], N_EXPERTS), lambda i: (0, 0)),
            pl.BlockSpec((1, N_EXPERTS), lambda i: (0, 0)),
        ],
        out_specs=pl.BlockSpec((BLOCK_ROWS, N_EXPERTS), lambda i: (i, 0)),
        out_shape=jax.ShapeDtypeStruct((chunk_rows, N_EXPERTS), jnp.float32),
    )(x, W, b2)


def _sc_route(logits):
    rows = logits.shape[0]
    rpw = rows // NW
    groups = rpw // 8
    mesh = plsc.VectorSubcoreMesh(core_axis_name="c", subcore_axis_name="s",
                                  num_cores=NC, num_subcores=NS)

    @functools.partial(
        pl.kernel,
        out_type=[
            jax.ShapeDtypeStruct((rows, N_EXPERTS), jnp.float32),
            jax.ShapeDtypeStruct((rows * TOPK,), jnp.int32),
        ],
        mesh=mesh,
        scratch_types=[
            pltpu.VMEM((rpw, N_EXPERTS), jnp.float32),
            pltpu.VMEM((rpw * TOPK,), jnp.int32),
        ],
    )
    def route(logits_hbm, gates_hbm, idx_hbm, lg_v, idx_v):
        wid = lax.axis_index("s") * NC + lax.axis_index("c")
        base = wid * rpw
        pltpu.sync_copy(logits_hbm.at[pl.ds(base, rpw)], lg_v)
        lane = lax.iota(jnp.int32, 16)
        neg_inf = jnp.full((16,), -jnp.inf, jnp.float32)
        zero = jnp.zeros((16,), jnp.float32)

        dnums = lax.GatherDimensionNumbers(offset_dims=(),
                                           collapsed_slice_dims=(0,),
                                           start_index_map=(0,))

        def shuffle(v, idx):
            return lax.gather(v, idx[:, None], dnums, (1,),
                              mode=lax.GatherScatterMode.PROMISE_IN_BOUNDS)

        def lane_max(v):
            for k in (1, 2, 4, 8):
                v = jnp.maximum(v, shuffle(v, lane ^ k))
            return v

        def lane_min(v):
            for k in (1, 2, 4, 8):
                v = jnp.minimum(v, shuffle(v, lane ^ k))
            return v

        def argmax_first(v, m):
            return lane_min(jnp.where(v == m, lane, N_EXPERTS))

        def group_body(g, _):
            acc = jnp.zeros((16,), jnp.int32)
            for r in range(8):
                i = g * 8 + r
                v = lg_v[i]
                m1 = lane_max(v)
                i1 = argmax_first(v, m1)
                masked = jnp.where(lane == i1, neg_inf, v)
                m2 = lane_max(masked)
                i2 = argmax_first(masked, m2)
                e = jnp.exp(m2 - m1)
                g2 = e / (1.0 + e)
                g1 = 1.0 - g2
                lg_v[i] = jnp.where(lane == i1, g1,
                                    jnp.where(lane == i2, g2, zero))
                acc = jnp.where(lane == 2 * r, i1, acc)
                acc = jnp.where(lane == 2 * r + 1, i2, acc)
            idx_v[pl.ds(g * 16, 16)] = acc
            return 0

        lax.fori_loop(0, groups, group_body, 0)
        pltpu.sync_copy(lg_v, gates_hbm.at[pl.ds(base, rpw)])
        pltpu.sync_copy(idx_v, idx_hbm.at[pl.ds(base * TOPK, rpw * TOPK)])

    gates, idx_flat = route(logits)
    return gates, idx_flat


@jax.jit
def kernel(x, W, b):
    x = x.astype(jnp.float32)
    Wf = W.astype(jnp.float32)
    rows = x.shape[0]
    b2 = b.reshape(1, N_EXPERTS).astype(jnp.float32)
    gates_parts, idx_parts = [], []
    base_row = 0
    for chunk_rows in CHUNK_ROWS:
        logits_c = _tc_logits_chunk(x, Wf, b2, base_row, chunk_rows)
        gates_c, idx_c = _sc_route(logits_c)
        gates_parts.append(gates_c)
        idx_parts.append(idx_c)
        base_row += chunk_rows
    gates = jnp.concatenate(gates_parts, axis=0)
    idx = jnp.concatenate(idx_parts, axis=0).reshape(rows, TOPK)
    return gates, idx


# speedup vs baseline: 1.1032x; 1.0293x over previous
import functools

import jax
import jax.numpy as jnp
from jax import lax
from jax.experimental import pallas as pl
from jax.experimental.pallas import tpu as pltpu
from jax.experimental.pallas import tpu_sc as plsc

BLOCK_ROWS = 1024
N_EXPERTS = 16
TOPK = 2
NC, NS, LANES = 2, 16, 16
NW = NC * NS
CHUNK_ROWS = (8192, 8192)


def _logits_block(x_ref, w_ref, b_ref, logits_ref):
    logits_ref[...] = jnp.dot(x_ref[...], w_ref[...],
                              preferred_element_type=jnp.float32) + b_ref[...]


def _tc_logits_chunk(x, W, b2, base_row, chunk_rows):
    blocks = chunk_rows // BLOCK_ROWS
    base_block = base_row // BLOCK_ROWS
    return pl.pallas_call(
        _logits_block,
        grid=(blocks,),
        in_specs=[
            pl.BlockSpec((BLOCK_ROWS, x.shape[1]),
                         lambda i: (base_block + i, 0)),
            pl.BlockSpec((x.shape[1], N_EXPERTS), lambda i: (0, 0)),
            pl.BlockSpec((1, N_EXPERTS), lambda i: (0, 0)),
        ],
        out_specs=pl.BlockSpec((BLOCK_ROWS, N_EXPERTS), lambda i: (i, 0)),
        out_shape=jax.ShapeDtypeStruct((chunk_rows, N_EXPERTS), jnp.float32),
    )(x, W, b2)


def _sc_route(logits):
    rows = logits.shape[0]
    rpw = rows // NW
    groups = rpw // 8
    mesh = plsc.VectorSubcoreMesh(core_axis_name="c", subcore_axis_name="s",
                                  num_cores=NC, num_subcores=NS)

    @functools.partial(
        pl.kernel,
        out_type=[
            jax.ShapeDtypeStruct((rows, N_EXPERTS), jnp.float32),
            jax.ShapeDtypeStruct((rows * TOPK,), jnp.int32),
        ],
        mesh=mesh,
        scratch_types=[
            pltpu.VMEM((rpw, N_EXPERTS), jnp.float32),
            pltpu.VMEM((rpw * TOPK,), jnp.int32),
        ],
    )
    def route(logits_hbm, gates_hbm, idx_hbm, lg_v, idx_v):
        wid = lax.axis_index("s") * NC + lax.axis_index("c")
        base = wid * rpw
        pltpu.sync_copy(logits_hbm.at[pl.ds(base, rpw)], lg_v)
        lane = lax.iota(jnp.int32, 16)
        neg_inf = jnp.full((16,), -jnp.inf, jnp.float32)
        zero = jnp.zeros((16,), jnp.float32)

        dnums = lax.GatherDimensionNumbers(offset_dims=(),
                                           collapsed_slice_dims=(0,),
                                           start_index_map=(0,))

        def shuffle(v, idx):
            return lax.gather(v, idx[:, None], dnums, (1,),
                              mode=lax.GatherScatterMode.PROMISE_IN_BOUNDS)

        def lane_max(v):
            for k in (1, 2, 4, 8):
                v = jnp.maximum(v, shuffle(v, lane ^ k))
            return v

        def lane_min(v):
            for k in (1, 2, 4, 8):
                v = jnp.minimum(v, shuffle(v, lane ^ k))
            return v

        def argmax_first(v, m):
            return lane_min(jnp.where(v == m, lane, N_EXPERTS))

        def group_body(g, _):
            acc = jnp.zeros((16,), jnp.int32)
            for r in range(8):
                i = g * 8 + r
                v = lg_v[i]
                m1 = lane_max(v)
                i1 = argmax_first(v, m1)
                masked = jnp.where(lane == i1, neg_inf, v)
                m2 = lane_max(masked)
                i2 = argmax_first(masked, m2)
                e = jnp.exp(m2 - m1)
                g2 = e / (1.0 + e)
                g1 = 1.0 - g2
                lg_v[i] = jnp.where(lane == i1, g1,
                                    jnp.where(lane == i2, g2, zero))
                acc = jnp.where(lane == 2 * r, i1, acc)
                acc = jnp.where(lane == 2 * r + 1, i2, acc)
            idx_v[pl.ds(g * 16, 16)] = acc
            return 0

        lax.fori_loop(0, groups, group_body, 0)
        pltpu.sync_copy(lg_v, gates_hbm.at[pl.ds(base, rpw)])
        pltpu.sync_copy(idx_v, idx_hbm.at[pl.ds(base * TOPK, rpw * TOPK)])

    gates, idx_flat = route(logits)
    return gates, idx_flat


@jax.jit
def kernel(x, W, b):
    x = x.astype(jnp.float32)
    Wf = W.astype(jnp.float32)
    rows = x.shape[0]
    b2 = b.reshape(1, N_EXPERTS).astype(jnp.float32)
    gates_parts, idx_parts = [], []
    base_row = 0
    for chunk_rows in CHUNK_ROWS:
        logits_c = _tc_logits_chunk(x, Wf, b2, base_row, chunk_rows)
        gates_c, idx_c = _sc_route(logits_c)
        gates_parts.append(gates_c)
        idx_parts.append(idx_c)
        base_row += chunk_rows
    gates = jnp.concatenate(gates_parts, axis=0)
    idx = jnp.concatenate(idx_parts, axis=0).reshape(rows, TOPK)
    return gates, idx
